# Initial kernel scaffold; baseline (speedup 1.0000x reference)
import functools

import jax
import jax.numpy as jnp
from jax import lax
from jax.experimental import pallas as pl
from jax.experimental.pallas import tpu as pltpu
from jax.experimental.pallas import tpu_sc as plsc

NC = 2
NS = 16
NW = NC * NS
CHUNK = 128


def _make_lookup(n, D, n_chunks):
    mesh = plsc.VectorSubcoreMesh(
        core_axis_name="c", subcore_axis_name="s",
        num_cores=NC, num_subcores=NS,
    )

    @functools.partial(
        pl.kernel,
        mesh=mesh,
        out_type=jax.ShapeDtypeStruct((n, D), jnp.float32),
        scratch_types=[
            pltpu.VMEM((n_chunks, CHUNK), jnp.int32),
            pltpu.VMEM((CHUNK, D), jnp.float32),
            pltpu.SemaphoreType.DMA,
        ],
    )
    def run(idx_hbm, table_hbm, out_hbm, idx_v, rows_v, gsem):
        wid = lax.axis_index("s") * NC + lax.axis_index("c")
        base = wid * (n_chunks * CHUNK)
        pltpu.sync_copy(idx_hbm.at[wid], idx_v)

        def body(j, _):
            pltpu.async_copy(table_hbm.at[idx_v.at[j]], rows_v, gsem).wait()
            pltpu.sync_copy(rows_v, out_hbm.at[pl.ds(base + j * CHUNK, CHUNK)])
            return 0

        lax.fori_loop(0, n_chunks, body, 0)

    return run


def kernel(input_tokens, table):
    B, H = input_tokens.shape
    V, D = table.shape
    n = B * H
    assert n % (NW * CHUNK) == 0
    n_chunks = n // (NW * CHUNK)
    idx = input_tokens.reshape(NW, n_chunks, CHUNK).astype(jnp.int32)
    out = _make_lookup(n, D, n_chunks)(idx, table)
    return out.reshape(B, H, D)


# speedup vs baseline: 1.6842x; 1.6842x over previous
import functools

import jax
import jax.numpy as jnp
from jax import lax
from jax.experimental import pallas as pl
from jax.experimental.pallas import tpu as pltpu
from jax.experimental.pallas import tpu_sc as plsc

NC = 2
NS = 16
NW = NC * NS
CHUNK = 128


def _make_lookup(n, D, n_chunks):
    mesh = plsc.VectorSubcoreMesh(
        core_axis_name="c", subcore_axis_name="s",
        num_cores=NC, num_subcores=NS,
    )

    @functools.partial(
        pl.kernel,
        mesh=mesh,
        out_type=jax.ShapeDtypeStruct((n, D), jnp.float32),
        scratch_types=[
            pltpu.VMEM((n_chunks, CHUNK), jnp.int32),
            pltpu.VMEM((CHUNK, D), jnp.float32),
            pltpu.SemaphoreType.DMA,
        ],
        compiler_params=pltpu.CompilerParams(use_tc_tiling_on_sc=False),
    )
    def run(idx_hbm, table_hbm, out_hbm, idx_v, rows_v, gsem):
        wid = lax.axis_index("s") * NC + lax.axis_index("c")
        base = wid * (n_chunks * CHUNK)
        pltpu.sync_copy(idx_hbm.at[wid], idx_v)

        def body(j, _):
            pltpu.async_copy(table_hbm.at[idx_v.at[j]], rows_v, gsem).wait()
            pltpu.sync_copy(rows_v, out_hbm.at[pl.ds(base + j * CHUNK, CHUNK)])
            return 0

        lax.fori_loop(0, n_chunks, body, 0)

    return run


def kernel(input_tokens, table):
    B, H = input_tokens.shape
    V, D = table.shape
    n = B * H
    assert n % (NW * CHUNK) == 0
    n_chunks = n // (NW * CHUNK)
    idx = input_tokens.reshape(NW, n_chunks, CHUNK).astype(jnp.int32)
    out = _make_lookup(n, D, n_chunks)(idx, table)
    return out.reshape(B, H, D)


# speedup vs baseline: 1.8766x; 1.1143x over previous
import functools

import jax
import jax.numpy as jnp
from jax import lax
from jax.experimental import pallas as pl
from jax.experimental.pallas import tpu as pltpu
from jax.experimental.pallas import tpu_sc as plsc

NC = 2
NS = 16
NW = NC * NS
CHUNK = 128
NBUF = 10


def _make_lookup(n, D, n_chunks):
    assert n_chunks % NBUF == 0
    n_groups = n_chunks // NBUF
    mesh = plsc.VectorSubcoreMesh(
        core_axis_name="c", subcore_axis_name="s",
        num_cores=NC, num_subcores=NS,
    )

    @functools.partial(
        pl.kernel,
        mesh=mesh,
        out_type=jax.ShapeDtypeStruct((n, D), jnp.float32),
        scratch_types=[
            pltpu.VMEM((n_chunks, CHUNK), jnp.int32),
            pltpu.VMEM((NBUF, CHUNK, D), jnp.float32),
            pltpu.SemaphoreType.DMA((NBUF,)),
            pltpu.SemaphoreType.DMA((NBUF,)),
        ],
        compiler_params=pltpu.CompilerParams(use_tc_tiling_on_sc=False),
    )
    def run(idx_hbm, table_hbm, out_hbm, idx_v, rows_v, gsem, ssem):
        wid = lax.axis_index("s") * NC + lax.axis_index("c")
        base = wid * (n_chunks * CHUNK)
        pltpu.sync_copy(idx_hbm.at[wid], idx_v)

        def gather_start(j, b):
            pltpu.async_copy(table_hbm.at[idx_v.at[j]], rows_v.at[b], gsem.at[b])

        def gather_wait(b):
            pltpu.make_async_copy(
                out_hbm.at[pl.ds(base, CHUNK)], rows_v.at[b], gsem.at[b]
            ).wait()

        def store_start(j, b):
            pltpu.async_copy(
                rows_v.at[b], out_hbm.at[pl.ds(base + j * CHUNK, CHUNK)], ssem.at[b]
            )

        def store_wait(b):
            pltpu.make_async_copy(
                out_hbm.at[pl.ds(base, CHUNK)], rows_v.at[b], ssem.at[b]
            ).wait()

        for b in range(NBUF):
            gather_start(b, b)

        def group(g, _):
            for b in range(NBUF):
                j = g * NBUF + b
                gather_wait(b)
                store_start(j, b)
                store_wait(b)
                gather_start(j + NBUF, b)
            return 0

        lax.fori_loop(0, n_groups - 1, group, 0)

        for b in range(NBUF):
            j = (n_groups - 1) * NBUF + b
            gather_wait(b)
            store_start(j, b)
            store_wait(b)

    return run


def kernel(input_tokens, table):
    B, H = input_tokens.shape
    V, D = table.shape
    n = B * H
    assert n % (NW * CHUNK) == 0
    n_chunks = n // (NW * CHUNK)
    idx = input_tokens.reshape(NW, n_chunks, CHUNK).astype(jnp.int32)
    out = _make_lookup(n, D, n_chunks)(idx, table)
    return out.reshape(B, H, D)


# speedup vs baseline: 1.8792x; 1.0014x over previous
import functools

import jax
import jax.numpy as jnp
from jax import lax
from jax.experimental import pallas as pl
from jax.experimental.pallas import tpu as pltpu
from jax.experimental.pallas import tpu_sc as plsc

NC = 2
NS = 16
NW = NC * NS
CHUNK = 128
NBUF = 10


def _make_lookup(B, H, D):
    bpw = B // NW
    kph = bpw // CHUNK
    n_iters = H * kph
    assert B % (NW * CHUNK) == 0 and n_iters % NBUF == 0
    n_groups = n_iters // NBUF
    mesh = plsc.VectorSubcoreMesh(
        core_axis_name="c", subcore_axis_name="s",
        num_cores=NC, num_subcores=NS,
    )

    @functools.partial(
        pl.kernel,
        mesh=mesh,
        out_type=jax.ShapeDtypeStruct((B, H, D), jnp.float32),
        scratch_types=[
            pltpu.VMEM((H, bpw), jnp.int32),
            pltpu.VMEM((NBUF, CHUNK, D), jnp.float32),
            pltpu.SemaphoreType.DMA((NBUF,)),
            pltpu.SemaphoreType.DMA((NBUF,)),
        ],
        compiler_params=pltpu.CompilerParams(use_tc_tiling_on_sc=False),
    )
    def run(idx_hbm, table_hbm, out_hbm, idx_v, rows_v, gsem, ssem):
        wid = lax.axis_index("s") * NC + lax.axis_index("c")
        b0 = wid * bpw
        pltpu.sync_copy(idx_hbm.at[:, pl.ds(b0, bpw)], idx_v)

        def gather_start(j, b):
            h = j // kph
            k = j % kph
            pltpu.async_copy(
                table_hbm.at[idx_v.at[h, pl.ds(k * CHUNK, CHUNK)]],
                rows_v.at[b],
                gsem.at[b],
            )

        def gather_wait(b):
            pltpu.make_async_copy(
                table_hbm.at[pl.ds(0, CHUNK)], rows_v.at[b], gsem.at[b]
            ).wait()

        def store_start(j, b):
            h = j // kph
            k = j % kph
            pltpu.async_copy(
                rows_v.at[b],
                out_hbm.at[pl.ds(b0 + k * CHUNK, CHUNK), h],
                ssem.at[b],
            )

        def store_wait(b):
            pltpu.make_async_copy(
                table_hbm.at[pl.ds(0, CHUNK)], rows_v.at[b], ssem.at[b]
            ).wait()

        for b in range(NBUF):
            gather_start(b, b)

        def group(g, _):
            for b in range(NBUF):
                j = g * NBUF + b
                gather_wait(b)
                store_start(j, b)
                store_wait(b)
                gather_start(j + NBUF, b)
            return 0

        lax.fori_loop(0, n_groups - 1, group, 0)

        for b in range(NBUF):
            j = (n_groups - 1) * NBUF + b
            gather_wait(b)
            store_start(j, b)
            store_wait(b)

    return run


def kernel(input_tokens, table):
    B, H = input_tokens.shape
    V, D = table.shape
    idx_t = input_tokens.T.astype(jnp.int32)
    return _make_lookup(B, H, D)(idx_t, table)
